# Initial kernel scaffold; baseline (speedup 1.0000x reference)
import jax
import jax.numpy as jnp
from jax.experimental import pallas as pl


def kernel(x, y):
    raise NotImplementedError("write your pallas kernel here")


# speedup vs baseline: 2178.5599x; 2178.5599x over previous
import functools

import jax
import jax.numpy as jnp
from jax import lax
from jax.experimental import pallas as pl
from jax.experimental.pallas import tpu as pltpu
from jax.experimental.pallas import tpu_sc as plsc

N = 2048
ROWS = 256
NC, NS, L = 2, 16, 16
NW = NC * NS
RPW = ROWS // NW
EPS = 1e-8
DT = 1e-3
GROUPS = N // L
BS_STEPS = 12


def _min_body(x_ref, y_ref, o_ref):
    m = jnp.minimum(jnp.min(x_ref[...]), jnp.min(y_ref[...]))
    o_ref[...] = jnp.full((1, 1), m, jnp.float32)


def _shift_transpose_body(m_ref, x_ref, y_ref, xt_ref, yt_ref):
    m = m_ref[0, 0]
    xt_ref[...] = jnp.transpose(x_ref[...], (0, 2, 1)) - m + EPS
    yt_ref[...] = jnp.transpose(y_ref[...], (0, 2, 1)) - m + EPS


def _cumsum_normalize(ref):
    def chunk(i, carry):
        cs = plsc.cumsum(ref[pl.ds(i * L, L)]) + carry
        ref[pl.ds(i * L, L)] = cs
        return jnp.max(cs)
    total = lax.fori_loop(0, GROUPS, chunk, jnp.float32(0.0))
    inv = jnp.ones((L,), jnp.float32) / jnp.full((L,), total, jnp.float32)

    def scale(i, _):
        ref[pl.ds(i * L, L)] = ref[pl.ds(i * L, L)] * inv
        return 0
    lax.fori_loop(0, GROUPS, scale, 0)


def _direction(q_ref, t_ref, right):
    lanes = lax.iota(jnp.int32, L)

    def group(g, acc):
        q = q_ref[pl.ds(g * L, L)]
        lo = jnp.zeros((L,), jnp.int32)
        hi = jnp.full((L,), N, jnp.int32)

        def step(_, lohi):
            lo, hi = lohi
            mid = lax.shift_right_logical(lo + hi, 1)
            tv = plsc.load_gather(t_ref, [jnp.minimum(mid, N - 1)])
            pred = (tv <= q) if right else (tv < q)
            return jnp.where(pred, mid + 1, lo), jnp.where(pred, hi, mid)

        c, _ = lax.fori_loop(0, BS_STEPS, step, (lo, hi))
        tprev = plsc.load_gather(t_ref, [jnp.maximum(c - 1, 0)])
        tprev = jnp.where(c > 0, tprev, 0.0)
        ivec = g * L + lanes
        qprev = plsc.load_gather(q_ref, [jnp.maximum(ivec - 1, 0)])
        qprev = jnp.where(ivec > 0, qprev, 0.0)
        delta = q - jnp.maximum(qprev, tprev)
        di = (ivec - jnp.minimum(c, N - 1)).astype(jnp.float32)
        return acc + delta * di * di

    return lax.fori_loop(0, GROUPS, group, jnp.zeros((L,), jnp.float32))


def _sc_body(xt_hbm, yt_hbm, out_hbm, a_v, b_v, w_v):
    wid = lax.axis_index("s") * NC + lax.axis_index("c")
    lanes = lax.iota(jnp.int32, L)

    def row_body(r, wvec):
        row = wid * RPW + r
        pltpu.sync_copy(xt_hbm.at[pl.ds(row * N, N)], a_v)
        pltpu.sync_copy(yt_hbm.at[pl.ds(row * N, N)], b_v)
        _cumsum_normalize(a_v)
        _cumsum_normalize(b_v)
        acc = _direction(a_v, b_v, right=False) + _direction(b_v, a_v, right=True)
        w = jnp.sum(acc) * (DT * DT)
        return jnp.where(lanes == r, w, wvec)

    wvec = lax.fori_loop(0, RPW, row_body, jnp.zeros((L,), jnp.float32))
    w_v[...] = wvec
    pltpu.sync_copy(w_v, out_hbm.at[wid])


def _sum_body(p_ref, o_ref):
    o_ref[...] = jnp.full((1, 1), jnp.sum(p_ref[...]), jnp.float32)


@jax.jit
def kernel(x, y):
    nb, nt, nr, nc = x.shape
    x2 = x.reshape(nb, nt, nr * nc)
    y2 = y.reshape(nb, nt, nr * nc)

    m = pl.pallas_call(
        _min_body,
        out_shape=jax.ShapeDtypeStruct((1, 1), jnp.float32),
    )(x2, y2)

    xt, yt = pl.pallas_call(
        _shift_transpose_body,
        grid=(nb,),
        in_specs=[
            pl.BlockSpec((1, 1), lambda b: (0, 0)),
            pl.BlockSpec((1, nt, nr * nc), lambda b: (b, 0, 0)),
            pl.BlockSpec((1, nt, nr * nc), lambda b: (b, 0, 0)),
        ],
        out_specs=[
            pl.BlockSpec((1, nr * nc, nt), lambda b: (b, 0, 0)),
            pl.BlockSpec((1, nr * nc, nt), lambda b: (b, 0, 0)),
        ],
        out_shape=[
            jax.ShapeDtypeStruct((nb, nr * nc, nt), jnp.float32),
            jax.ShapeDtypeStruct((nb, nr * nc, nt), jnp.float32),
        ],
    )(m, x2, y2)

    mesh = plsc.VectorSubcoreMesh(
        core_axis_name="c", subcore_axis_name="s", num_cores=NC, num_subcores=NS
    )
    partials = pl.kernel(
        _sc_body,
        out_type=jax.ShapeDtypeStruct((NW, L), jnp.float32),
        mesh=mesh,
        compiler_params=pltpu.CompilerParams(needs_layout_passes=False),
        scratch_types=[
            pltpu.VMEM((N,), jnp.float32),
            pltpu.VMEM((N,), jnp.float32),
            pltpu.VMEM((L,), jnp.float32),
        ],
    )(xt.reshape(ROWS * N), yt.reshape(ROWS * N))

    loss = pl.pallas_call(
        _sum_body,
        out_shape=jax.ShapeDtypeStruct((1, 1), jnp.float32),
    )(partials.reshape(4, 128))
    return loss[0, 0]


# speedup vs baseline: 2401.0989x; 1.1021x over previous
import functools

import jax
import jax.numpy as jnp
from jax import lax
from jax.experimental import pallas as pl
from jax.experimental.pallas import tpu as pltpu
from jax.experimental.pallas import tpu_sc as plsc

N = 2048
ROWS = 256
NC, NS, L = 2, 16, 16
NW = NC * NS
RPW = ROWS // NW
EPS = 1e-8
DT = 1e-3
GROUPS = N // L
BS_STEPS = 12


def _min_body(x_ref, y_ref, o_ref):
    m = jnp.minimum(jnp.min(x_ref[...]), jnp.min(y_ref[...]))
    o_ref[...] = jnp.full((1, 1), m, jnp.float32)


def _shift_transpose_body(m_ref, x_ref, y_ref, xt_ref, yt_ref):
    m = m_ref[0, 0]
    xt_ref[...] = jnp.transpose(x_ref[...], (0, 2, 1)) - m + EPS
    yt_ref[...] = jnp.transpose(y_ref[...], (0, 2, 1)) - m + EPS


def _cumsum_inplace(ref):
    def chunk2(i, carry):
        c0 = ref[pl.ds((2 * i) * L, L)]
        c1 = ref[pl.ds((2 * i + 1) * L, L)]
        s0 = plsc.cumsum(c0)
        s1 = plsc.cumsum(c1)
        cs0 = s0 + carry
        carry1 = jnp.max(cs0)
        cs1 = s1 + carry1
        ref[pl.ds((2 * i) * L, L)] = cs0
        ref[pl.ds((2 * i + 1) * L, L)] = cs1
        return jnp.max(cs1)
    return lax.fori_loop(0, GROUPS // 2, chunk2, jnp.float32(0.0))


_SHIFT_IDX = None


def _direction(q_ref, t_ref, r_qt, r_tq, right):
    lanes = lax.iota(jnp.int32, L)
    shift_idx = jnp.maximum(lanes - 1, 0)
    dnums = lax.GatherDimensionNumbers(
        offset_dims=(), collapsed_slice_dims=(0,), start_index_map=(0,)
    )

    def _lane_shift(q):
        return lax.gather(
            q, shift_idx[:, None], dnums, slice_sizes=(1,),
            mode=lax.GatherScatterMode.PROMISE_IN_BOUNDS,
        )

    def one_group(g, carry_q):
        q = q_ref[pl.ds(g * L, L)]
        qs = q * r_qt
        lo = jnp.zeros((L,), jnp.int32)
        hi = jnp.full((L,), N, jnp.int32)

        def step(_, lohi):
            lo, hi = lohi
            mid = lax.shift_right_logical(lo + hi, 1)
            tv = plsc.load_gather(t_ref, [jnp.minimum(mid, N - 1)])
            pred = (tv <= qs) if right else (tv < qs)
            return jnp.where(pred, mid + 1, lo), jnp.where(pred, hi, mid)

        c, _ = lax.fori_loop(0, BS_STEPS, step, (lo, hi))
        tprev = plsc.load_gather(t_ref, [jnp.maximum(c - 1, 0)])
        tprev = jnp.where(c > 0, tprev * r_tq, 0.0)
        qprev = _lane_shift(q)
        qprev = jnp.where(lanes == 0, carry_q, qprev)
        ivec = g * L + lanes
        delta = q - jnp.maximum(qprev, tprev)
        di = (ivec - jnp.minimum(c, N - 1)).astype(jnp.float32)
        return delta * di * di, jnp.max(q)

    def group2(i, state):
        acc0, acc1, carry_q = state
        w0, carry_q = one_group(2 * i, carry_q)
        w1, carry_q = one_group(2 * i + 1, carry_q)
        return acc0 + w0, acc1 + w1, carry_q

    z = jnp.zeros((L,), jnp.float32)
    acc0, acc1, _ = lax.fori_loop(
        0, GROUPS // 2, group2, (z, z, jnp.float32(0.0))
    )
    return acc0 + acc1


def _sc_body(xt_hbm, yt_hbm, out_hbm, a_v, b_v, w_v):
    wid = lax.axis_index("s") * NC + lax.axis_index("c")
    lanes = lax.iota(jnp.int32, L)
    ones = jnp.ones((L,), jnp.float32)

    def row_body(r, wvec):
        row = wid * RPW + r
        pltpu.sync_copy(xt_hbm.at[pl.ds(row * N, N)], a_v)
        pltpu.sync_copy(yt_hbm.at[pl.ds(row * N, N)], b_v)
        ta = _cumsum_inplace(a_v)
        tb = _cumsum_inplace(b_v)
        ta_v = jnp.full((L,), ta, jnp.float32)
        tb_v = jnp.full((L,), tb, jnp.float32)
        r_ab = tb_v / ta_v
        r_ba = ta_v / tb_v
        acc = (_direction(a_v, b_v, r_ab, r_ba, right=False) / ta_v
               + _direction(b_v, a_v, r_ba, r_ab, right=True) / tb_v)
        w = jnp.sum(acc) * (DT * DT)
        return jnp.where(lanes == r, w, wvec)

    wvec = lax.fori_loop(0, RPW, row_body, jnp.zeros((L,), jnp.float32))
    w_v[...] = wvec
    pltpu.sync_copy(w_v, out_hbm.at[wid])


def _sum_body(p_ref, o_ref):
    o_ref[...] = jnp.full((1, 1), jnp.sum(p_ref[...]), jnp.float32)


@jax.jit
def kernel(x, y):
    nb, nt, nr, nc = x.shape
    x2 = x.reshape(nb, nt, nr * nc)
    y2 = y.reshape(nb, nt, nr * nc)

    m = pl.pallas_call(
        _min_body,
        out_shape=jax.ShapeDtypeStruct((1, 1), jnp.float32),
    )(x2, y2)

    xt, yt = pl.pallas_call(
        _shift_transpose_body,
        grid=(nb,),
        in_specs=[
            pl.BlockSpec((1, 1), lambda b: (0, 0)),
            pl.BlockSpec((1, nt, nr * nc), lambda b: (b, 0, 0)),
            pl.BlockSpec((1, nt, nr * nc), lambda b: (b, 0, 0)),
        ],
        out_specs=[
            pl.BlockSpec((1, nr * nc, nt), lambda b: (b, 0, 0)),
            pl.BlockSpec((1, nr * nc, nt), lambda b: (b, 0, 0)),
        ],
        out_shape=[
            jax.ShapeDtypeStruct((nb, nr * nc, nt), jnp.float32),
            jax.ShapeDtypeStruct((nb, nr * nc, nt), jnp.float32),
        ],
    )(m, x2, y2)

    mesh = plsc.VectorSubcoreMesh(
        core_axis_name="c", subcore_axis_name="s", num_cores=NC, num_subcores=NS
    )
    partials = pl.kernel(
        _sc_body,
        out_type=jax.ShapeDtypeStruct((NW, L), jnp.float32),
        mesh=mesh,
        compiler_params=pltpu.CompilerParams(needs_layout_passes=False),
        scratch_types=[
            pltpu.VMEM((N,), jnp.float32),
            pltpu.VMEM((N,), jnp.float32),
            pltpu.VMEM((L,), jnp.float32),
        ],
    )(xt.reshape(ROWS * N), yt.reshape(ROWS * N))

    loss = pl.pallas_call(
        _sum_body,
        out_shape=jax.ShapeDtypeStruct((1, 1), jnp.float32),
    )(partials.reshape(4, 128))
    return loss[0, 0]
